# Initial kernel scaffold; baseline (speedup 1.0000x reference)
import jax
import jax.numpy as jnp
from jax.experimental import pallas as pl


def kernel(data, offsets, weight):
    raise NotImplementedError("write your pallas kernel here")


# speedup vs baseline: 28.5137x; 28.5137x over previous
import functools

import jax
import jax.numpy as jnp
from jax import lax
from jax.experimental import pallas as pl
from jax.experimental.pallas import tpu as pltpu
from jax.experimental.pallas import tpu_sc as plsc


def kernel(data, offsets, weight):
    NNZ = data.shape[0]
    B = offsets.shape[0] - 1
    NE, D = weight.shape
    NC, NS = 2, 16
    NW = NC * NS
    BPW = B // NW
    C = 512
    NQ = C // 128
    SLAB = BPW + 1
    L = 16

    mesh = plsc.VectorSubcoreMesh(core_axis_name="c", subcore_axis_name="s")

    @functools.partial(
        pl.kernel,
        out_type=jax.ShapeDtypeStruct((B, D), jnp.float32),
        mesh=mesh,
        scratch_types=[
            pltpu.VMEM((BPW + 16,), jnp.int32),
            pltpu.VMEM((C,), jnp.int32),
            pltpu.VMEM((NQ, 128), jnp.int32),
            pltpu.VMEM((C,), jnp.int32),
            pltpu.VMEM((C, D), jnp.float32),
            pltpu.VMEM_SHARED((NS * SLAB, D), jnp.float32),
            pltpu.SemaphoreType.DMA,
        ],
        compiler_params=pltpu.CompilerParams(
            needs_layout_passes=False, use_tc_tiling_on_sc=False),
    )
    def emb_bag(data_h, offs_h, weight_h, out_h,
                off_v, idx_v, seg_v, hist_v, rows_v, acc_sh, sem):
        cid = lax.axis_index("c")
        sid = lax.axis_index("s")
        w = cid * NS + sid
        b0 = w * BPW
        slab0 = sid * SLAB
        dummy = slab0 + BPW

        pltpu.sync_copy(offs_h.at[pl.ds(b0, BPW + 1)],
                        off_v.at[pl.ds(0, BPW + 1)])
        e0 = off_v[pl.ds(0, L)][0]
        e1 = jnp.where(w == NW - 1, NNZ, off_v[pl.ds(BPW, L)][0])
        base_a = (e0 // 8) * 8

        zf = jnp.zeros((L,), jnp.float32)
        zi = jnp.zeros((L,), jnp.int32)
        ones = jnp.ones((L,), jnp.int32)
        iota = lax.iota(jnp.int32, L)

        def zrow(r, carry):
            for d in range(D // L):
                rows_v[r, pl.ds(d * L, L)] = zf
            return carry
        lax.fori_loop(0, C, zrow, 0)
        pltpu.sync_copy(rows_v, acc_sh.at[pl.ds(slab0, C)])
        pltpu.sync_copy(rows_v.at[pl.ds(0, 1)],
                        acc_sh.at[pl.ds(slab0 + BPW, 1)])

        nchunks = (e1 - base_a + C - 1) // C

        def chunk(kk, run):
            s_k = base_a + kk * C
            base_k = jnp.minimum(s_k, NNZ - C)
            pltpu.sync_copy(data_h.at[pl.ds(base_k, C)], idx_v)
            cps = [
                pltpu.async_copy(
                    weight_h.at[idx_v.at[pl.ds(q * 128, 128)]],
                    rows_v.at[pl.ds(q * 128, 128)], sem)
                for q in range(NQ)
            ]
            for j in range(C // L):
                hist_v[pl.ds(j * L, L)] = zi
            for j in range(BPW // L):
                ov = off_v[pl.ds(j * L, L)]
                m = (ov >= s_k) & (ov - base_k < C)
                plsc.addupdate_scatter(hist_v, [ov - base_k], ones, mask=m)
            lo = jnp.maximum(s_k, e0)
            r = run
            for j in range(C // L):
                h = hist_v[pl.ds(j * L, L)]
                cs = plsc.cumsum(h)
                p = base_k + j * L + iota
                valid = (p >= lo) & (p < e1)
                seg = jnp.where(valid, slab0 + r + cs - 1, dummy)
                seg_v[j // 8, pl.ds((j % 8) * L, L)] = seg
                r = r + jnp.sum(h)
            for cp in cps:
                cp.wait()
            for q in range(NQ):
                pltpu.sync_copy(rows_v.at[pl.ds(q * 128, 128)],
                                acc_sh.at[seg_v.at[q]], add=True)
            return r

        lax.fori_loop(0, nchunks, chunk, jnp.int32(0))

        pltpu.sync_copy(acc_sh.at[pl.ds(slab0, BPW)], rows_v)

        def fin(g, carry):
            b = g * L
            o0v = off_v[pl.ds(b, L)]
            o1v = off_v[pl.ds(b + 1, L)]
            o1v = jnp.where(b + iota == BPW - 1, e1, o1v)
            cntv = (o1v - o0v).astype(jnp.float32)
            scv = jnp.where(cntv > 0.0, 1.0 / cntv, 0.0)
            for i in range(L):
                sv = jnp.full((L,), scv[i], jnp.float32)
                for d in range(D // L):
                    rows_v[b + i, pl.ds(d * L, L)] = (
                        rows_v[b + i, pl.ds(d * L, L)] * sv)
            return carry
        lax.fori_loop(0, BPW // L, fin, 0)

        pltpu.sync_copy(rows_v, out_h.at[pl.ds(b0, BPW)])

    return emb_bag(data, offsets, weight)
